# Initial kernel scaffold; baseline (speedup 1.0000x reference)
import jax
import jax.numpy as jnp
from jax.experimental import pallas as pl


def kernel(tokens, embed_weights, embed_bias):
    raise NotImplementedError("write your pallas kernel here")


# speedup vs baseline: 1.3547x; 1.3547x over previous
import functools
import math

import jax
import jax.numpy as jnp
from jax import lax
from jax.experimental import pallas as pl
from jax.experimental.pallas import tpu as pltpu
from jax.experimental.pallas import tpu_sc as plsc

D_MODEL = 128
LANES = 16
GROUPS = D_MODEL // LANES
NUM_CORES = 2
NUM_SUBCORES = 16
NW = NUM_CORES * NUM_SUBCORES
SCALE = math.sqrt(D_MODEL)


@functools.partial(jax.jit, static_argnums=(3, 4))
def _embed_sc(tokens_tiled, weights, bias, n_chunks, chunk):
    b_per_w = n_chunks * chunk
    total = NW * b_per_w
    mesh = plsc.VectorSubcoreMesh(core_axis_name="c", subcore_axis_name="s")

    @functools.partial(
        pl.kernel,
        mesh=mesh,
        out_type=jax.ShapeDtypeStruct((total, D_MODEL), jnp.float32),
        scratch_types=[
            pltpu.VMEM((n_chunks, chunk), jnp.int32),
            pltpu.VMEM((2, chunk, D_MODEL), jnp.float32),
            pltpu.VMEM((D_MODEL,), jnp.float32),
            pltpu.SemaphoreType.DMA,
            pltpu.SemaphoreType.DMA,
        ],
    )
    def k(tok_hbm, tab_hbm, bias_hbm, out_hbm, idx_v, rows_v, bias_v, sem0, sem1):
        wid = lax.axis_index("s") * NUM_CORES + lax.axis_index("c")
        pltpu.sync_copy(tok_hbm.at[wid], idx_v)
        pltpu.sync_copy(bias_hbm, bias_v)
        bias_regs = [bias_v[pl.ds(j * LANES, LANES)] for j in range(GROUPS)]
        sems = [sem0, sem1]

        copies = [None, None]
        copies[0] = pltpu.async_copy(tab_hbm.at[idx_v.at[0]], rows_v.at[0], sems[0])
        for g in range(n_chunks):
            b = g & 1
            if g + 1 < n_chunks:
                nb = (g + 1) & 1
                copies[nb] = pltpu.async_copy(
                    tab_hbm.at[idx_v.at[g + 1]], rows_v.at[nb], sems[nb]
                )
            copies[b].wait()

            def body(r, carry):
                for j in range(GROUPS):
                    sl = pl.ds(j * LANES, LANES)
                    rows_v[b, r, sl] = rows_v[b, r, sl] * SCALE + bias_regs[j]
                return carry

            lax.fori_loop(0, chunk, body, 0)
            pltpu.sync_copy(
                rows_v.at[b],
                out_hbm.at[pl.ds(wid * b_per_w + g * chunk, chunk)],
            )

    return k(tokens_tiled, weights, bias)


def kernel(tokens, embed_weights, embed_bias):
    b, s = tokens.shape
    total = b * s
    chunk = 128
    b_per_w = total // NW
    n_chunks = b_per_w // chunk
    tok = tokens.reshape(NW, n_chunks, chunk).astype(jnp.int32)
    out = _embed_sc(tok, embed_weights, embed_bias, n_chunks, chunk)
    return out.reshape(b, s, D_MODEL)


# speedup vs baseline: 1.4184x; 1.0470x over previous
import functools
import math

import jax
import jax.numpy as jnp
from jax import lax
from jax.experimental import pallas as pl
from jax.experimental.pallas import tpu as pltpu
from jax.experimental.pallas import tpu_sc as plsc

D_MODEL = 128
LANES = 16
GROUPS = D_MODEL // LANES
NUM_CORES = 2
NUM_SUBCORES = 16
NW = NUM_CORES * NUM_SUBCORES
SCALE = math.sqrt(D_MODEL)


@functools.partial(jax.jit, static_argnums=(3, 4))
def _embed_sc(tokens_tiled, weights, bias, n_chunks, chunk):
    b_per_w = n_chunks * chunk
    total = NW * b_per_w
    mesh = plsc.VectorSubcoreMesh(core_axis_name="c", subcore_axis_name="s")

    nbuf = 4

    @functools.partial(
        pl.kernel,
        mesh=mesh,
        out_type=jax.ShapeDtypeStruct((total, D_MODEL), jnp.float32),
        scratch_types=[
            pltpu.VMEM((n_chunks, chunk), jnp.int32),
            pltpu.VMEM((nbuf, chunk, D_MODEL), jnp.float32),
            pltpu.VMEM((D_MODEL,), jnp.float32),
        ]
        + [pltpu.SemaphoreType.DMA] * (2 * nbuf),
    )
    def k(tok_hbm, tab_hbm, bias_hbm, out_hbm, idx_v, rows_v, bias_v, *sems):
        gsems, ssems = sems[:nbuf], sems[nbuf:]
        wid = lax.axis_index("s") * NUM_CORES + lax.axis_index("c")
        pltpu.sync_copy(tok_hbm.at[wid], idx_v)
        pltpu.sync_copy(bias_hbm, bias_v)
        bias_regs = [bias_v[pl.ds(j * LANES, LANES)] for j in range(GROUPS)]

        def start_gather(g):
            b = g % nbuf
            return pltpu.async_copy(tab_hbm.at[idx_v.at[g]], rows_v.at[b], gsems[b])

        gather_h = [None] * n_chunks
        store_h = [None] * n_chunks
        store_waited = [False] * n_chunks
        for g in range(min(nbuf - 1, n_chunks)):
            gather_h[g] = start_gather(g)

        for g in range(n_chunks):
            b = g % nbuf
            ng = g + nbuf - 1
            if ng < n_chunks:
                prev = ng - nbuf
                if prev >= 0 and not store_waited[prev]:
                    store_h[prev].wait()
                    store_waited[prev] = True
                gather_h[ng] = start_gather(ng)
            gather_h[g].wait()

            def body(r, carry):
                for j in range(GROUPS):
                    sl = pl.ds(j * LANES, LANES)
                    rows_v[b, r, sl] = rows_v[b, r, sl] * SCALE + bias_regs[j]
                return carry

            lax.fori_loop(0, chunk, body, 0)
            store_h[g] = pltpu.async_copy(
                rows_v.at[b],
                out_hbm.at[pl.ds(wid * b_per_w + g * chunk, chunk)],
                ssems[b],
            )

        for g in range(n_chunks):
            if store_h[g] is not None and not store_waited[g]:
                store_h[g].wait()

    return k(tokens_tiled, weights, bias)


def kernel(tokens, embed_weights, embed_bias):
    b, s = tokens.shape
    total = b * s
    chunk = 128
    b_per_w = total // NW
    n_chunks = b_per_w // chunk
    tok = tokens.reshape(NW, n_chunks, chunk).astype(jnp.int32)
    out = _embed_sc(tok, embed_weights, embed_bias, n_chunks, chunk)
    return out.reshape(b, s, D_MODEL)
